# Initial kernel scaffold; baseline (speedup 1.0000x reference)
import jax
import jax.numpy as jnp
from jax.experimental import pallas as pl


def kernel(x, incidence_index, W1, b1, W2, b2, eps):
    raise NotImplementedError("write your pallas kernel here")


# speedup vs baseline: 7.0506x; 7.0506x over previous
import functools

import jax
import jax.numpy as jnp
from jax import lax
from jax.experimental import pallas as pl
from jax.experimental.pallas import tpu as pltpu
from jax.experimental.pallas import tpu_sc as plsc

N = 10000
C = 10000
E = 320000
D = 128

NC = 2
NS = 16
NW = NC * NS

K = 128
NCHUNK = E // K
FULL = NCHUNK // NW
REM = NCHUNK % NW
ROWS_PER_TILE = 632
CPAD = NS * ROWS_PER_TILE


def _sc_body(x_hbm, inc_hbm, zeros_hbm, out_hbm, src_v, dst_v, rows_v, acc, sem):
    cid = lax.axis_index("c")
    sid = lax.axis_index("s")
    wid = sid * NC + cid

    r0 = sid * ROWS_PER_TILE
    pltpu.sync_copy(zeros_hbm, acc.at[pl.ds(r0, ROWS_PER_TILE)])
    plsc.subcore_barrier()

    nj = FULL + jnp.where(wid < REM, 1, 0).astype(jnp.int32)

    def step(j, carry):
        base = (j * NW + wid) * K
        pltpu.sync_copy(inc_hbm.at[0, pl.ds(base, K)], src_v)
        pltpu.sync_copy(inc_hbm.at[1, pl.ds(base, K)], dst_v)
        pltpu.async_copy(x_hbm.at[src_v], rows_v, sem).wait()
        pltpu.sync_copy(rows_v, acc.at[dst_v], add=True)
        return carry

    lax.fori_loop(0, nj, step, jnp.int32(0))
    plsc.subcore_barrier()

    pltpu.sync_copy(acc.at[pl.ds(r0, ROWS_PER_TILE)],
                    out_hbm.at[cid, pl.ds(r0, ROWS_PER_TILE)])


_sc_segment_sum = functools.partial(
    pl.kernel,
    out_type=jax.ShapeDtypeStruct((NC, CPAD, D), jnp.float32),
    mesh=plsc.VectorSubcoreMesh(
        core_axis_name="c", subcore_axis_name="s", num_cores=NC, num_subcores=NS
    ),
    scratch_types=[
        pltpu.VMEM((K,), jnp.int32),
        pltpu.VMEM((K,), jnp.int32),
        pltpu.VMEM((K, D), jnp.float32),
        pltpu.VMEM_SHARED((CPAD, D), jnp.float32),
        pltpu.SemaphoreType.DMA,
    ],
)(_sc_body)


BC = 2000


def _mlp_body(p_ref, w1_ref, b1_ref, w2_ref, b2_ref, o_ref):
    a = p_ref[0] + p_ref[1]
    h = jnp.dot(a, w1_ref[...], preferred_element_type=jnp.float32) + b1_ref[...]
    h = jnp.maximum(h, 0.0)
    o_ref[...] = (
        jnp.dot(h, w2_ref[...], preferred_element_type=jnp.float32) + b2_ref[...]
    )


_mlp = pl.pallas_call(
    _mlp_body,
    grid=(C // BC,),
    in_specs=[
        pl.BlockSpec((NC, BC, D), lambda i: (0, i, 0)),
        pl.BlockSpec((D, D), lambda i: (0, 0)),
        pl.BlockSpec((1, D), lambda i: (0, 0)),
        pl.BlockSpec((D, D), lambda i: (0, 0)),
        pl.BlockSpec((1, D), lambda i: (0, 0)),
    ],
    out_specs=pl.BlockSpec((BC, D), lambda i: (i, 0)),
    out_shape=jax.ShapeDtypeStruct((C, D), jnp.float32),
)


def kernel(x, incidence_index, W1, b1, W2, b2, eps):
    inc = incidence_index.astype(jnp.int32)
    zeros = jnp.zeros((ROWS_PER_TILE, D), dtype=jnp.float32)
    partials = _sc_segment_sum(x, inc, zeros)
    w1s = W1 * (1.0 + eps)
    return _mlp(partials, w1s, b1.reshape(1, D), W2, b2.reshape(1, D))
